# baseline (speedup 1.0000x reference)
import jax
import jax.numpy as jnp
from jax.experimental import pallas as pl

_BM = 512


def _masked_matmul_kernel(x_ref, table_ref, o_ref):
    mask = (x_ref[...] != 0).astype(jnp.float32)
    o_ref[...] = jnp.dot(mask, table_ref[...],
                         preferred_element_type=jnp.float32)


@jax.jit
def kernel(x, table):
    batch, num_cat = x.shape
    _, embed_dim = table.shape
    return pl.pallas_call(
        _masked_matmul_kernel,
        grid=(batch // _BM,),
        in_specs=[
            pl.BlockSpec((_BM, num_cat), lambda i: (i, 0)),
            pl.BlockSpec((num_cat, embed_dim), lambda i: (0, 0)),
        ],
        out_specs=pl.BlockSpec((_BM, embed_dim), lambda i: (i, 0)),
        out_shape=jax.ShapeDtypeStruct((batch, embed_dim), jnp.float32),
    )(x, table)
